# baseline (speedup 1.0000x reference)
import jax
import jax.numpy as jnp
from jax.experimental import pallas as pl
from jax.experimental.pallas import tpu as pltpu

_BS = 128
_NBUF = 8


def _pipelined_kernel(x_hbm, t_hbm, o_hbm, xb, tb, ob, sx, st, so):
    seq_len, batch, _ = x_hbm.shape
    nsteps = seq_len // _BS

    def in_copies(i):
        slot = i % _NBUF
        return (
            pltpu.make_async_copy(
                x_hbm.at[pl.ds(i * _BS, _BS)], xb.at[slot], sx.at[slot]),
            pltpu.make_async_copy(
                t_hbm.at[pl.ds(i * _BS, _BS)], tb.at[slot], st.at[slot]),
        )

    def out_copy(i):
        slot = i % _NBUF
        return pltpu.make_async_copy(
            ob.at[slot], o_hbm.at[pl.ds(i * _BS, _BS)], so.at[slot])

    for i in range(min(_NBUF, nsteps)):
        for c in in_copies(i):
            c.start()

    for i in range(nsteps):
        slot = i % _NBUF
        for c in in_copies(i):
            c.wait()
        if i >= _NBUF:
            out_copy(i - _NBUF).wait()
        t = tb[slot]
        for b in range(batch):
            ob[slot, :, b, :] = xb[slot, :, b, :] + t
        out_copy(i).start()
        if i + _NBUF < nsteps:
            for c in in_copies(i + _NBUF):
                c.start()

    for i in range(max(0, nsteps - _NBUF), nsteps):
        out_copy(i).wait()


def kernel(x, table):
    seq_len, batch, d = x.shape
    return pl.pallas_call(
        _pipelined_kernel,
        in_specs=[
            pl.BlockSpec(memory_space=pl.ANY),
            pl.BlockSpec(memory_space=pl.ANY),
        ],
        out_specs=pl.BlockSpec(memory_space=pl.ANY),
        out_shape=jax.ShapeDtypeStruct((seq_len, batch, d), x.dtype),
        scratch_shapes=[
            pltpu.VMEM((_NBUF, _BS, batch, d), x.dtype),
            pltpu.VMEM((_NBUF, _BS, d), table.dtype),
            pltpu.VMEM((_NBUF, _BS, batch, d), x.dtype),
            pltpu.SemaphoreType.DMA((_NBUF,)),
            pltpu.SemaphoreType.DMA((_NBUF,)),
            pltpu.SemaphoreType.DMA((_NBUF,)),
        ],
    )(x, table)


# speedup vs baseline: 1.0005x; 1.0005x over previous
import jax
import jax.numpy as jnp
from jax.experimental import pallas as pl
from jax.experimental.pallas import tpu as pltpu

_BS = 256
_NBUF = 6


def _pipelined_kernel(x_hbm, t_hbm, o_hbm, xb, tb, ob, sx, st, so):
    seq_len, batch, _ = x_hbm.shape
    nsteps = seq_len // _BS

    def in_copies(i):
        slot = i % _NBUF
        return (
            pltpu.make_async_copy(
                x_hbm.at[pl.ds(i * _BS, _BS)], xb.at[slot], sx.at[slot]),
            pltpu.make_async_copy(
                t_hbm.at[pl.ds(i * _BS, _BS)], tb.at[slot], st.at[slot]),
        )

    def out_copy(i):
        slot = i % _NBUF
        return pltpu.make_async_copy(
            ob.at[slot], o_hbm.at[pl.ds(i * _BS, _BS)], so.at[slot])

    for i in range(min(_NBUF, nsteps)):
        for c in in_copies(i):
            c.start()

    for i in range(nsteps):
        slot = i % _NBUF
        for c in in_copies(i):
            c.wait()
        if i >= _NBUF:
            out_copy(i - _NBUF).wait()
        t = tb[slot]
        for b in range(batch):
            ob[slot, :, b, :] = xb[slot, :, b, :] + t
        out_copy(i).start()
        if i + _NBUF < nsteps:
            for c in in_copies(i + _NBUF):
                c.start()

    for i in range(max(0, nsteps - _NBUF), nsteps):
        out_copy(i).wait()


def kernel(x, table):
    seq_len, batch, d = x.shape
    return pl.pallas_call(
        _pipelined_kernel,
        in_specs=[
            pl.BlockSpec(memory_space=pl.ANY),
            pl.BlockSpec(memory_space=pl.ANY),
        ],
        out_specs=pl.BlockSpec(memory_space=pl.ANY),
        out_shape=jax.ShapeDtypeStruct((seq_len, batch, d), x.dtype),
        scratch_shapes=[
            pltpu.VMEM((_NBUF, _BS, batch, d), x.dtype),
            pltpu.VMEM((_NBUF, _BS, d), table.dtype),
            pltpu.VMEM((_NBUF, _BS, batch, d), x.dtype),
            pltpu.SemaphoreType.DMA((_NBUF,)),
            pltpu.SemaphoreType.DMA((_NBUF,)),
            pltpu.SemaphoreType.DMA((_NBUF,)),
        ],
    )(x, table)


# speedup vs baseline: 1.0104x; 1.0099x over previous
import jax
import jax.numpy as jnp
from jax.experimental import pallas as pl
from jax.experimental.pallas import tpu as pltpu

_BS = 64
_NBUF = 12


def _pipelined_kernel(x_hbm, t_hbm, o_hbm, xb, tb, ob, sx, st, so):
    seq_len, batch, _ = x_hbm.shape
    nsteps = seq_len // _BS

    def in_copies(i):
        slot = i % _NBUF
        return (
            pltpu.make_async_copy(
                x_hbm.at[pl.ds(i * _BS, _BS)], xb.at[slot], sx.at[slot]),
            pltpu.make_async_copy(
                t_hbm.at[pl.ds(i * _BS, _BS)], tb.at[slot], st.at[slot]),
        )

    def out_copy(i):
        slot = i % _NBUF
        return pltpu.make_async_copy(
            ob.at[slot], o_hbm.at[pl.ds(i * _BS, _BS)], so.at[slot])

    for i in range(min(_NBUF, nsteps)):
        for c in in_copies(i):
            c.start()

    for i in range(nsteps):
        slot = i % _NBUF
        for c in in_copies(i):
            c.wait()
        if i >= _NBUF:
            out_copy(i - _NBUF).wait()
        t = tb[slot]
        for b in range(batch):
            ob[slot, :, b, :] = xb[slot, :, b, :] + t
        out_copy(i).start()
        if i + _NBUF < nsteps:
            for c in in_copies(i + _NBUF):
                c.start()

    for i in range(max(0, nsteps - _NBUF), nsteps):
        out_copy(i).wait()


def kernel(x, table):
    seq_len, batch, d = x.shape
    return pl.pallas_call(
        _pipelined_kernel,
        in_specs=[
            pl.BlockSpec(memory_space=pl.ANY),
            pl.BlockSpec(memory_space=pl.ANY),
        ],
        out_specs=pl.BlockSpec(memory_space=pl.ANY),
        out_shape=jax.ShapeDtypeStruct((seq_len, batch, d), x.dtype),
        scratch_shapes=[
            pltpu.VMEM((_NBUF, _BS, batch, d), x.dtype),
            pltpu.VMEM((_NBUF, _BS, d), table.dtype),
            pltpu.VMEM((_NBUF, _BS, batch, d), x.dtype),
            pltpu.SemaphoreType.DMA((_NBUF,)),
            pltpu.SemaphoreType.DMA((_NBUF,)),
            pltpu.SemaphoreType.DMA((_NBUF,)),
        ],
    )(x, table)


# speedup vs baseline: 1.0116x; 1.0012x over previous
import jax
import jax.numpy as jnp
from jax.experimental import pallas as pl
from jax.experimental.pallas import tpu as pltpu

_BS = 64
_NBUF = 16


def _pipelined_kernel(x_hbm, t_hbm, o_hbm, xb, tb, ob, sx, st, so):
    seq_len, batch, _ = x_hbm.shape
    nsteps = seq_len // _BS

    def in_copies(i):
        slot = i % _NBUF
        return (
            pltpu.make_async_copy(
                x_hbm.at[pl.ds(i * _BS, _BS)], xb.at[slot], sx.at[slot]),
            pltpu.make_async_copy(
                t_hbm.at[pl.ds(i * _BS, _BS)], tb.at[slot], st.at[slot]),
        )

    def out_copy(i):
        slot = i % _NBUF
        return pltpu.make_async_copy(
            ob.at[slot], o_hbm.at[pl.ds(i * _BS, _BS)], so.at[slot])

    for i in range(min(_NBUF, nsteps)):
        for c in in_copies(i):
            c.start()

    for i in range(nsteps):
        slot = i % _NBUF
        for c in in_copies(i):
            c.wait()
        if i >= _NBUF:
            out_copy(i - _NBUF).wait()
        t = tb[slot]
        for b in range(batch):
            ob[slot, :, b, :] = xb[slot, :, b, :] + t
        out_copy(i).start()
        if i + _NBUF < nsteps:
            for c in in_copies(i + _NBUF):
                c.start()

    for i in range(max(0, nsteps - _NBUF), nsteps):
        out_copy(i).wait()


def kernel(x, table):
    seq_len, batch, d = x.shape
    return pl.pallas_call(
        _pipelined_kernel,
        in_specs=[
            pl.BlockSpec(memory_space=pl.ANY),
            pl.BlockSpec(memory_space=pl.ANY),
        ],
        out_specs=pl.BlockSpec(memory_space=pl.ANY),
        out_shape=jax.ShapeDtypeStruct((seq_len, batch, d), x.dtype),
        scratch_shapes=[
            pltpu.VMEM((_NBUF, _BS, batch, d), x.dtype),
            pltpu.VMEM((_NBUF, _BS, d), table.dtype),
            pltpu.VMEM((_NBUF, _BS, batch, d), x.dtype),
            pltpu.SemaphoreType.DMA((_NBUF,)),
            pltpu.SemaphoreType.DMA((_NBUF,)),
            pltpu.SemaphoreType.DMA((_NBUF,)),
        ],
    )(x, table)
